# baseline (speedup 1.0000x reference)
import functools

import jax
import jax.numpy as jnp
from jax import lax
from jax.experimental import pallas as pl
from jax.experimental.pallas import tpu as pltpu
from jax.experimental.pallas import tpu_sc as plsc

_N = 50000
_D = 256
_EPS = 1e-12
_RB = 16
_BW = _RB * _D
_NBLK = _N // _RB
_NW = 32
_MAXITER = -(-_NBLK // _NW)


def _rsqrt_nr(s):
    i = lax.bitcast_convert_type(s, jnp.int32)
    i = jnp.int32(0x5F3759DF) - lax.shift_right_logical(i, 1)
    r = lax.bitcast_convert_type(i, jnp.float32)
    for _ in range(3):
        r = r * (1.5 - 0.5 * s * r * r)
    return r


def _pair_contrib(sp, sq, d):
    rp = jnp.where(sp >= _EPS * _EPS, _rsqrt_nr(sp), 1.0 / _EPS)
    rq = jnp.where(sq >= _EPS * _EPS, _rsqrt_nr(sq), 1.0 / _EPS)
    return sp * rp * rp + sq * rq * rq - 2.0 * d * rp * rq


def _block_contrib(b1, b2, b3, b4):
    rows = lax.iota(jnp.int32, 16) * _D
    zeros = jnp.zeros((16,), jnp.float32)
    unroll = 8

    def jbody(jo, carry):
        s11, s22, d12, s33, s44, d34 = carry
        idx0 = rows + jo * unroll
        for k in range(unroll):
            idx = idx0 + k
            v1 = plsc.load_gather(b1, [idx])
            v2 = plsc.load_gather(b2, [idx])
            v3 = plsc.load_gather(b3, [idx])
            v4 = plsc.load_gather(b4, [idx])
            s11 = s11 + v1 * v1
            s22 = s22 + v2 * v2
            d12 = d12 + v1 * v2
            s33 = s33 + v3 * v3
            s44 = s44 + v4 * v4
            d34 = d34 + v3 * v4
        return s11, s22, d12, s33, s44, d34

    s11, s22, d12, s33, s44, d34 = lax.fori_loop(
        0, _D // unroll, jbody, (zeros,) * 6)
    return _pair_contrib(s11, s22, d12) + _pair_contrib(s33, s44, d34)


def _make_sc_call():
    mesh = plsc.VectorSubcoreMesh(core_axis_name="c", subcore_axis_name="s")

    @functools.partial(
        pl.kernel,
        mesh=mesh,
        compiler_params=pltpu.CompilerParams(needs_layout_passes=False),
        out_type=jax.ShapeDtypeStruct((_NW * 16,), jnp.float32),
        scratch_types=[
            pltpu.VMEM((_BW,), jnp.float32),
            pltpu.VMEM((_BW,), jnp.float32),
            pltpu.VMEM((_BW,), jnp.float32),
            pltpu.VMEM((_BW,), jnp.float32),
            pltpu.VMEM((16,), jnp.float32),
        ],
    )
    def sc_call(p1h, q2h, p2h, q1h, outh, b1, b2, b3, b4, accv):
        c = lax.axis_index("c")
        s = lax.axis_index("s")
        wid = s * 2 + c
        accv[...] = jnp.zeros((16,), jnp.float32)

        def blk_body(i, _):
            blk = wid + i * _NW

            @pl.when(blk < _NBLK)
            def _():
                base = blk * _BW
                pltpu.sync_copy(p1h.at[pl.ds(base, _BW)], b1)
                pltpu.sync_copy(q2h.at[pl.ds(base, _BW)], b2)
                pltpu.sync_copy(p2h.at[pl.ds(base, _BW)], b3)
                pltpu.sync_copy(q1h.at[pl.ds(base, _BW)], b4)
                accv[...] = accv[...] + _block_contrib(b1, b2, b3, b4)

            return 0

        lax.fori_loop(0, _MAXITER, blk_body, 0)
        pltpu.sync_copy(accv, outh.at[pl.ds(wid * 16, 16)])

    return sc_call


_sc_call = _make_sc_call()


def kernel(pred1, proj2, pred2, proj1):
    partials = _sc_call(
        pred1.reshape(-1), proj2.reshape(-1),
        pred2.reshape(-1), proj1.reshape(-1))
    return 0.5 * jnp.sum(partials)


# speedup vs baseline: 1.3166x; 1.3166x over previous
import functools

import jax
import jax.numpy as jnp
from jax import lax
from jax.experimental import pallas as pl
from jax.experimental.pallas import tpu as pltpu
from jax.experimental.pallas import tpu_sc as plsc

_N = 50000
_D = 256
_EPS = 1e-12
_RB = 16
_BW = _RB * _D
_NBLK = _N // _RB
_NW = 32
_MAXITER = -(-_NBLK // _NW)


def _rsqrt_nr(s):
    i = lax.bitcast_convert_type(s, jnp.int32)
    i = jnp.int32(0x5F3759DF) - lax.shift_right_logical(i, 1)
    r = lax.bitcast_convert_type(i, jnp.float32)
    for _ in range(3):
        r = r * (1.5 - 0.5 * s * r * r)
    return r


def _pair_contrib(sp, sq, d):
    rp = jnp.where(sp >= _EPS * _EPS, _rsqrt_nr(sp), 1.0 / _EPS)
    rq = jnp.where(sq >= _EPS * _EPS, _rsqrt_nr(sq), 1.0 / _EPS)
    return sp * rp * rp + sq * rq * rq - 2.0 * d * rp * rq


def _block_contrib(b1, b2, b3, b4):
    rows = lax.iota(jnp.int32, 16) * _D
    zeros = jnp.zeros((16,), jnp.float32)
    unroll = 8

    def jbody(jo, carry):
        s11, s22, d12, s33, s44, d34 = carry
        idx0 = rows + jo * unroll
        for k in range(unroll):
            idx = idx0 + k
            v1 = plsc.load_gather(b1, [idx])
            v2 = plsc.load_gather(b2, [idx])
            v3 = plsc.load_gather(b3, [idx])
            v4 = plsc.load_gather(b4, [idx])
            s11 = s11 + v1 * v1
            s22 = s22 + v2 * v2
            d12 = d12 + v1 * v2
            s33 = s33 + v3 * v3
            s44 = s44 + v4 * v4
            d34 = d34 + v3 * v4
        return s11, s22, d12, s33, s44, d34

    s11, s22, d12, s33, s44, d34 = lax.fori_loop(
        0, _D // unroll, jbody, (zeros,) * 6)
    return _pair_contrib(s11, s22, d12) + _pair_contrib(s33, s44, d34)


def _make_sc_call():
    mesh = plsc.VectorSubcoreMesh(core_axis_name="c", subcore_axis_name="s")

    @functools.partial(
        pl.kernel,
        mesh=mesh,
        compiler_params=pltpu.CompilerParams(needs_layout_passes=False),
        out_type=jax.ShapeDtypeStruct((_NW * 16,), jnp.float32),
        scratch_types=[
            pltpu.VMEM((_BW,), jnp.float32),
            pltpu.VMEM((_BW,), jnp.float32),
            pltpu.VMEM((_BW,), jnp.float32),
            pltpu.VMEM((_BW,), jnp.float32),
            pltpu.VMEM((_BW,), jnp.float32),
            pltpu.VMEM((_BW,), jnp.float32),
            pltpu.VMEM((_BW,), jnp.float32),
            pltpu.VMEM((_BW,), jnp.float32),
            pltpu.VMEM((16,), jnp.float32),
            pltpu.SemaphoreType.DMA,
            pltpu.SemaphoreType.DMA,
        ],
    )
    def sc_call(p1h, q2h, p2h, q1h, outh,
                a1, a2, a3, a4, b1, b2, b3, b4, accv, sem_a, sem_b):
        c = lax.axis_index("c")
        s = lax.axis_index("s")
        wid = s * 2 + c
        accv[...] = jnp.zeros((16,), jnp.float32)
        hbm = (p1h, q2h, p2h, q1h)
        slots = ((a1, a2, a3, a4, sem_a), (b1, b2, b3, b4, sem_b))

        def issue(i, slot):
            blk = wid + i * _NW

            @pl.when(blk < _NBLK)
            def _():
                base = blk * _BW
                for src, dst in zip(hbm, slot[:4]):
                    pltpu.async_copy(src.at[pl.ds(base, _BW)], dst, slot[4])

        def drain_compute(i, slot):
            blk = wid + i * _NW

            @pl.when(blk < _NBLK)
            def _():
                base = blk * _BW
                for src, dst in zip(hbm, slot[:4]):
                    pltpu.make_async_copy(
                        src.at[pl.ds(base, _BW)], dst, slot[4]).wait()
                accv[...] = accv[...] + _block_contrib(*slot[:4])

        issue(0, slots[0])

        def pair_body(i2, _):
            i = i2 * 2
            issue(i + 1, slots[1])
            drain_compute(i, slots[0])
            issue(i + 2, slots[0])
            drain_compute(i + 1, slots[1])
            return 0

        lax.fori_loop(0, _MAXITER // 2, pair_body, 0)
        pltpu.sync_copy(accv, outh.at[pl.ds(wid * 16, 16)])

    return sc_call


_sc_call = _make_sc_call()


def kernel(pred1, proj2, pred2, proj1):
    partials = _sc_call(
        pred1.reshape(-1), proj2.reshape(-1),
        pred2.reshape(-1), proj1.reshape(-1))
    return 0.5 * jnp.sum(partials)


# speedup vs baseline: 4.4669x; 3.3928x over previous
import functools

import jax
import jax.numpy as jnp
from jax import lax
from jax.experimental import pallas as pl
from jax.experimental.pallas import tpu as pltpu
from jax.experimental.pallas import tpu_sc as plsc

_N = 50000
_D = 256
_EPS = 1e-12
_RB = 16
_BW = _RB * _D
_NBLK = _N // _RB
_NW = 32
_MAXITER = -(-_NBLK // _NW)


def _rsqrt_nr(s):
    i = lax.bitcast_convert_type(s, jnp.int32)
    i = jnp.int32(0x5F3759DF) - lax.shift_right_logical(i, 1)
    r = lax.bitcast_convert_type(i, jnp.float32)
    for _ in range(3):
        r = r * (1.5 - 0.5 * s * r * r)
    return r


def _pair_contrib(sp, sq, d):
    rp = jnp.where(sp >= _EPS * _EPS, _rsqrt_nr(sp), 1.0 / _EPS)
    rq = jnp.where(sq >= _EPS * _EPS, _rsqrt_nr(sq), 1.0 / _EPS)
    return sp * rp * rp + sq * rq * rq - 2.0 * d * rp * rq


def _block_contrib(b1, b2, b3, b4):
    lanes = lax.iota(jnp.int32, 16)
    zeros = jnp.zeros((16,), jnp.float32)

    def rbody(r, carry):
        s11v, s22v, d12v, s33v, s44v, d34v = carry
        base = r * _D
        c11 = c22 = c12 = c33 = c44 = c34 = zeros
        for c in range(_D // 16):
            off = base + c * 16
            v1 = b1[pl.ds(off, 16)]
            v2 = b2[pl.ds(off, 16)]
            v3 = b3[pl.ds(off, 16)]
            v4 = b4[pl.ds(off, 16)]
            c11 = c11 + v1 * v1
            c22 = c22 + v2 * v2
            c12 = c12 + v1 * v2
            c33 = c33 + v3 * v3
            c44 = c44 + v4 * v4
            c34 = c34 + v3 * v4
        m = lanes == r
        s11v = jnp.where(m, jnp.sum(c11), s11v)
        s22v = jnp.where(m, jnp.sum(c22), s22v)
        d12v = jnp.where(m, jnp.sum(c12), d12v)
        s33v = jnp.where(m, jnp.sum(c33), s33v)
        s44v = jnp.where(m, jnp.sum(c44), s44v)
        d34v = jnp.where(m, jnp.sum(c34), d34v)
        return s11v, s22v, d12v, s33v, s44v, d34v

    s11v, s22v, d12v, s33v, s44v, d34v = lax.fori_loop(
        0, _RB, rbody, (zeros,) * 6)
    return _pair_contrib(s11v, s22v, d12v) + _pair_contrib(s33v, s44v, d34v)


def _make_sc_call():
    mesh = plsc.VectorSubcoreMesh(core_axis_name="c", subcore_axis_name="s")

    @functools.partial(
        pl.kernel,
        mesh=mesh,
        compiler_params=pltpu.CompilerParams(needs_layout_passes=False),
        out_type=jax.ShapeDtypeStruct((_NW * 16,), jnp.float32),
        scratch_types=[
            pltpu.VMEM((_BW,), jnp.float32),
            pltpu.VMEM((_BW,), jnp.float32),
            pltpu.VMEM((_BW,), jnp.float32),
            pltpu.VMEM((_BW,), jnp.float32),
            pltpu.VMEM((_BW,), jnp.float32),
            pltpu.VMEM((_BW,), jnp.float32),
            pltpu.VMEM((_BW,), jnp.float32),
            pltpu.VMEM((_BW,), jnp.float32),
            pltpu.VMEM((16,), jnp.float32),
            pltpu.SemaphoreType.DMA,
            pltpu.SemaphoreType.DMA,
        ],
    )
    def sc_call(p1h, q2h, p2h, q1h, outh,
                a1, a2, a3, a4, b1, b2, b3, b4, accv, sem_a, sem_b):
        c = lax.axis_index("c")
        s = lax.axis_index("s")
        wid = s * 2 + c
        accv[...] = jnp.zeros((16,), jnp.float32)
        hbm = (p1h, q2h, p2h, q1h)
        slots = ((a1, a2, a3, a4, sem_a), (b1, b2, b3, b4, sem_b))

        def issue(i, slot):
            blk = wid + i * _NW

            @pl.when(blk < _NBLK)
            def _():
                base = blk * _BW
                for src, dst in zip(hbm, slot[:4]):
                    pltpu.async_copy(src.at[pl.ds(base, _BW)], dst, slot[4])

        def drain_compute(i, slot):
            blk = wid + i * _NW

            @pl.when(blk < _NBLK)
            def _():
                base = blk * _BW
                for src, dst in zip(hbm, slot[:4]):
                    pltpu.make_async_copy(
                        src.at[pl.ds(base, _BW)], dst, slot[4]).wait()
                accv[...] = accv[...] + _block_contrib(*slot[:4])

        issue(0, slots[0])

        def pair_body(i2, _):
            i = i2 * 2
            issue(i + 1, slots[1])
            drain_compute(i, slots[0])
            issue(i + 2, slots[0])
            drain_compute(i + 1, slots[1])
            return 0

        lax.fori_loop(0, _MAXITER // 2, pair_body, 0)
        pltpu.sync_copy(accv, outh.at[pl.ds(wid * 16, 16)])

    return sc_call


_sc_call = _make_sc_call()


def kernel(pred1, proj2, pred2, proj1):
    partials = _sc_call(
        pred1.reshape(-1), proj2.reshape(-1),
        pred2.reshape(-1), proj1.reshape(-1))
    return 0.5 * jnp.sum(partials)


# speedup vs baseline: 4.7729x; 1.0685x over previous
import functools

import jax
import jax.numpy as jnp
from jax import lax
from jax.experimental import pallas as pl
from jax.experimental.pallas import tpu as pltpu
from jax.experimental.pallas import tpu_sc as plsc

_N = 50000
_D = 256
_EPS = 1e-12
_RB = 16
_BW = _RB * _D
_NBLK = _N // _RB
_NW = 32
_MAXITER = -(-_NBLK // _NW)


def _rsqrt_nr(s):
    i = lax.bitcast_convert_type(s, jnp.int32)
    i = jnp.int32(0x5F3759DF) - lax.shift_right_logical(i, 1)
    r = lax.bitcast_convert_type(i, jnp.float32)
    for _ in range(3):
        r = r * (1.5 - 0.5 * s * r * r)
    return r


def _pair_contrib(sp, sq, d):
    rp = jnp.where(sp >= _EPS * _EPS, _rsqrt_nr(sp), 1.0 / _EPS)
    rq = jnp.where(sq >= _EPS * _EPS, _rsqrt_nr(sq), 1.0 / _EPS)
    return sp * rp * rp + sq * rq * rq - 2.0 * d * rp * rq


def _block_contrib(b1, b2, b3, b4):
    lanes = lax.iota(jnp.int32, 16)
    zeros = jnp.zeros((16,), jnp.float32)

    def rbody(r, carry):
        s11v, s22v, d12v, s33v, s44v, d34v = carry
        base = r * _D
        c11 = c22 = c12 = c33 = c44 = c34 = zeros
        for c in range(_D // 16):
            off = base + c * 16
            v1 = b1[pl.ds(off, 16)]
            v2 = b2[pl.ds(off, 16)]
            v3 = b3[pl.ds(off, 16)]
            v4 = b4[pl.ds(off, 16)]
            c11 = c11 + v1 * v1
            c22 = c22 + v2 * v2
            c12 = c12 + v1 * v2
            c33 = c33 + v3 * v3
            c44 = c44 + v4 * v4
            c34 = c34 + v3 * v4
        m = lanes == r
        s11v = jnp.where(m, jnp.sum(c11), s11v)
        s22v = jnp.where(m, jnp.sum(c22), s22v)
        d12v = jnp.where(m, jnp.sum(c12), d12v)
        s33v = jnp.where(m, jnp.sum(c33), s33v)
        s44v = jnp.where(m, jnp.sum(c44), s44v)
        d34v = jnp.where(m, jnp.sum(c34), d34v)
        return s11v, s22v, d12v, s33v, s44v, d34v

    s11v, s22v, d12v, s33v, s44v, d34v = lax.fori_loop(
        0, _RB, rbody, (zeros,) * 6)
    return _pair_contrib(s11v, s22v, d12v) + _pair_contrib(s33v, s44v, d34v)


def _make_sc_call():
    mesh = plsc.VectorSubcoreMesh(core_axis_name="c", subcore_axis_name="s")

    @functools.partial(
        pl.kernel,
        mesh=mesh,
        compiler_params=pltpu.CompilerParams(needs_layout_passes=False),
        out_type=jax.ShapeDtypeStruct((_NW * 16,), jnp.float32),
        scratch_types=[
            pltpu.VMEM((_BW,), jnp.float32),
            pltpu.VMEM((_BW,), jnp.float32),
            pltpu.VMEM((_BW,), jnp.float32),
            pltpu.VMEM((_BW,), jnp.float32),
            pltpu.VMEM((_BW,), jnp.float32),
            pltpu.VMEM((_BW,), jnp.float32),
            pltpu.VMEM((_BW,), jnp.float32),
            pltpu.VMEM((_BW,), jnp.float32),
            pltpu.VMEM((_BW,), jnp.float32),
            pltpu.VMEM((_BW,), jnp.float32),
            pltpu.VMEM((_BW,), jnp.float32),
            pltpu.VMEM((_BW,), jnp.float32),
            pltpu.VMEM((_BW,), jnp.float32),
            pltpu.VMEM((_BW,), jnp.float32),
            pltpu.VMEM((_BW,), jnp.float32),
            pltpu.VMEM((_BW,), jnp.float32),
            pltpu.VMEM((16,), jnp.float32),
            pltpu.SemaphoreType.DMA,
            pltpu.SemaphoreType.DMA,
            pltpu.SemaphoreType.DMA,
            pltpu.SemaphoreType.DMA,
        ],
    )
    def sc_call(p1h, q2h, p2h, q1h, outh,
                a1, a2, a3, a4, b1, b2, b3, b4,
                c1, c2, c3, c4, d1, d2, d3, d4,
                accv, sem_a, sem_b, sem_c, sem_d):
        c = lax.axis_index("c")
        s = lax.axis_index("s")
        wid = s * 2 + c
        accv[...] = jnp.zeros((16,), jnp.float32)
        hbm = (p1h, q2h, p2h, q1h)
        slots = ((a1, a2, a3, a4, sem_a), (b1, b2, b3, b4, sem_b),
                 (c1, c2, c3, c4, sem_c), (d1, d2, d3, d4, sem_d))

        def issue(i, slot):
            blk = wid + i * _NW

            @pl.when(blk < _NBLK)
            def _():
                base = blk * _BW
                for src, dst in zip(hbm, slot[:4]):
                    pltpu.async_copy(src.at[pl.ds(base, _BW)], dst, slot[4])

        def drain_compute(i, slot):
            blk = wid + i * _NW

            @pl.when(blk < _NBLK)
            def _():
                base = blk * _BW
                for src, dst in zip(hbm, slot[:4]):
                    pltpu.make_async_copy(
                        src.at[pl.ds(base, _BW)], dst, slot[4]).wait()
                accv[...] = accv[...] + _block_contrib(*slot[:4])

        issue(0, slots[0])
        issue(1, slots[1])
        issue(2, slots[2])

        def ring_body(i4, _):
            i = i4 * 4
            issue(i + 3, slots[3])
            drain_compute(i, slots[0])
            issue(i + 4, slots[0])
            drain_compute(i + 1, slots[1])
            issue(i + 5, slots[1])
            drain_compute(i + 2, slots[2])
            issue(i + 6, slots[2])
            drain_compute(i + 3, slots[3])
            return 0

        lax.fori_loop(0, -(-_MAXITER // 4), ring_body, 0)
        pltpu.sync_copy(accv, outh.at[pl.ds(wid * 16, 16)])

    return sc_call


_sc_call = _make_sc_call()


def kernel(pred1, proj2, pred2, proj1):
    partials = _sc_call(
        pred1.reshape(-1), proj2.reshape(-1),
        pred2.reshape(-1), proj1.reshape(-1))
    return 0.5 * jnp.sum(partials)
